# Initial kernel scaffold; baseline (speedup 1.0000x reference)
import jax
import jax.numpy as jnp
from jax.experimental import pallas as pl


def kernel(h, e, edge_index, params):
    raise NotImplementedError("write your pallas kernel here")


# speedup vs baseline: 1.2758x; 1.2758x over previous
import functools

import jax
import jax.numpy as jnp
from jax import lax
from jax.experimental import pallas as pl
from jax.experimental.pallas import tpu as pltpu
from jax.experimental.pallas import tpu_sc as plsc

N_NODES = 10000
N_EDGES = 320000
HID = 128
HALF = 64
CHUNK = 128
NCHUNK = N_EDGES // CHUNK
NSUB = 16
ZUNIT = 16
NZUNIT = N_NODES // ZUNIT
EBLK = 4000
NEB = N_EDGES // EBLK
F32 = jnp.float32

@functools.cache
def _sc_mesh():
    return plsc.VectorSubcoreMesh(core_axis_name="c", subcore_axis_name="s",
                                  num_cores=2, num_subcores=NSUB)


def _mm_body(x_ref, w_ref, b_ref, *out_refs, widths):
    y = jnp.dot(x_ref[...], w_ref[...], preferred_element_type=F32) + b_ref[...]
    off = 0
    for r, w in zip(out_refs, widths):
        r[...] = y[:, off:off + w]
        off += w


def _node_mm(x, w, b, widths):
    n = x.shape[0]
    out = pl.pallas_call(
        functools.partial(_mm_body, widths=widths),
        out_shape=[jax.ShapeDtypeStruct((n, wd), F32) for wd in widths],
    )(x, w, b.reshape(1, -1))
    return out


def _emm_body(x_ref, w_ref, b_ref, o_ref):
    o_ref[...] = (jnp.dot(x_ref[...], w_ref[...], preferred_element_type=F32)
                  + b_ref[...])


def _edge_mm(x, w, b):
    din, dout = w.shape
    return pl.pallas_call(
        _emm_body,
        grid=(NEB,),
        in_specs=[pl.BlockSpec((EBLK, din), lambda j: (j, 0)),
                  pl.BlockSpec((din, dout), lambda j: (0, 0)),
                  pl.BlockSpec((1, dout), lambda j: (0, 0))],
        out_specs=pl.BlockSpec((EBLK, dout), lambda j: (j, 0)),
        out_shape=jax.ShapeDtypeStruct((N_EDGES, dout), F32),
    )(x, w, b.reshape(1, -1))


def _estats_body(en_ref, sum_ref, sq_ref):
    j = pl.program_id(0)
    en = jnp.concatenate([en_ref[0, :, :HALF], en_ref[1, :, HALF:]], axis=-1)

    @pl.when(j == 0)
    def _():
        sum_ref[...] = jnp.zeros_like(sum_ref)
        sq_ref[...] = jnp.zeros_like(sq_ref)

    sum_ref[...] += jnp.sum(en, axis=0, keepdims=True)
    sq_ref[...] += jnp.sum(en * en, axis=0, keepdims=True)


def _estats(enew2):
    return pl.pallas_call(
        _estats_body,
        grid=(NEB,),
        in_specs=[pl.BlockSpec((2, EBLK, HID), lambda j: (0, j, 0))],
        out_specs=[pl.BlockSpec((1, HID), lambda j: (0, 0)),
                   pl.BlockSpec((1, HID), lambda j: (0, 0))],
        out_shape=[jax.ShapeDtypeStruct((1, HID), F32),
                   jax.ShapeDtypeStruct((1, HID), F32)],
    )(enew2)


def _eapply_body(en_ref, ein_ref, sum_ref, sq_ref, g_ref, b_ref, o_ref):
    en = jnp.concatenate([en_ref[0, :, :HALF], en_ref[1, :, HALF:]], axis=-1)
    mu = sum_ref[...] / N_EDGES
    var = sq_ref[...] / N_EDGES - mu * mu
    xn = g_ref[...] * (en - mu) / jnp.sqrt(var + 1e-5) + b_ref[...]
    o_ref[...] = ein_ref[...] + jnp.maximum(xn, 0.0)


def _eapply(enew2, e_in, ssum, ssq, g, b):
    return pl.pallas_call(
        _eapply_body,
        grid=(NEB,),
        in_specs=[pl.BlockSpec((2, EBLK, HID), lambda j: (0, j, 0)),
                  pl.BlockSpec((EBLK, HID), lambda j: (j, 0)),
                  pl.BlockSpec((1, HID), lambda j: (0, 0)),
                  pl.BlockSpec((1, HID), lambda j: (0, 0)),
                  pl.BlockSpec((1, HID), lambda j: (0, 0)),
                  pl.BlockSpec((1, HID), lambda j: (0, 0))],
        out_specs=pl.BlockSpec((EBLK, HID), lambda j: (j, 0)),
        out_shape=jax.ShapeDtypeStruct((N_EDGES, HID), F32),
    )(enew2, e_in, ssum, ssq, g.reshape(1, -1), b.reshape(1, -1))


def _hfin_body(ah_ref, nd_ref, hin_ref, g_ref, b_ref, *rest, assign):
    num = jnp.concatenate([nd_ref[0, :, :HALF], nd_ref[1, :, :HALF]], axis=-1)
    den = jnp.concatenate([nd_ref[0, :, HALF:], nd_ref[1, :, HALF:]], axis=-1)
    hn = ah_ref[...] + num / (den + 1e-6)
    mu = jnp.mean(hn, axis=0, keepdims=True)
    var = jnp.mean((hn - mu) ** 2, axis=0, keepdims=True)
    hn = g_ref[...] * (hn - mu) / jnp.sqrt(var + 1e-5) + b_ref[...]
    hn = hin_ref[...] + jnp.maximum(hn, 0.0)
    if assign:
        sw_ref, sb_ref, o_ref, s_ref = rest
        o_ref[...] = hn
        logits = jnp.dot(hn, sw_ref[...], preferred_element_type=F32) + sb_ref[...]
        s_ref[...] = jax.nn.softmax(logits, axis=-1)
    else:
        (o_ref,) = rest
        o_ref[...] = hn


def _h_fin(ah, nd, h_in, g, b, sw=None, sb=None):
    assign = sw is not None
    if assign:
        out_shape = [jax.ShapeDtypeStruct((N_NODES, HID), F32),
                     jax.ShapeDtypeStruct((N_NODES, sw.shape[1]), F32)]
        args = (ah, nd, h_in, g.reshape(1, -1), b.reshape(1, -1),
                sw, sb.reshape(1, -1))
    else:
        out_shape = jax.ShapeDtypeStruct((N_NODES, HID), F32)
        args = (ah, nd, h_in, g.reshape(1, -1), b.reshape(1, -1))
    return pl.pallas_call(
        functools.partial(_hfin_body, assign=assign),
        out_shape=out_shape,
    )(*args)


def _head_body(y1_ref, b1_ref, w2_ref, b2_ref, w3_ref, b3_ref, o_ref):
    y = y1_ref[...] + b1_ref[...]
    y = jnp.maximum(y, 0.0)
    y = jnp.maximum(jnp.dot(y, w2_ref[...], preferred_element_type=F32)
                    + b2_ref[...], 0.0)
    o_ref[...] = jnp.dot(y, w3_ref[...], preferred_element_type=F32) + b3_ref[...]


def _head(y1, b1, w2, b2, w3, b3):
    d2 = w2.shape[1]
    do = w3.shape[1]
    return pl.pallas_call(
        _head_body,
        grid=(NEB,),
        in_specs=[pl.BlockSpec((EBLK, HID), lambda j: (j, 0)),
                  pl.BlockSpec((1, HID), lambda j: (0, 0)),
                  pl.BlockSpec((HID, d2), lambda j: (0, 0)),
                  pl.BlockSpec((1, d2), lambda j: (0, 0)),
                  pl.BlockSpec((d2, do), lambda j: (0, 0)),
                  pl.BlockSpec((1, do), lambda j: (0, 0))],
        out_specs=pl.BlockSpec((EBLK, do), lambda j: (j, 0)),
        out_shape=jax.ShapeDtypeStruct((N_EDGES, do), F32),
    )(y1, b1.reshape(1, -1), w2, b2.reshape(1, -1), w3, b3.reshape(1, -1))


def _edge_body(src_hbm, dst_hbm, dbh_hbm, eh_hbm, ce_hbm,
               enew_hbm, nd_hbm,
               srcv, gsrcv, dstw, ce_b, dbh_b, ct_b, acc):
    c = lax.axis_index("c")
    s = lax.axis_index("s")
    nunit = (NZUNIT - s + NSUB - 1) // NSUB

    def zrow(r, carry):
        for j in range(HID // 16):
            ct_b[r, pl.ds(j * 16, 16)] = jnp.zeros((16,), F32)
        return carry

    lax.fori_loop(0, ZUNIT, zrow, 0)

    def zero_unit(t, carry):
        row = (s + t * NSUB) * ZUNIT
        pltpu.sync_copy(ct_b.at[pl.ds(0, ZUNIT)], acc.at[pl.ds(row, ZUNIT)])
        return carry

    lax.fori_loop(0, nunit, zero_unit, 0)
    plsc.subcore_barrier()
    lo = (NCHUNK * s) // NSUB
    hi = (NCHUNK * (s + 1)) // NSUB

    def chunk(k, carry):
        base = k * CHUNK
        pltpu.sync_copy(src_hbm.at[pl.ds(base, CHUNK)], srcv)
        pltpu.sync_copy(dst_hbm.at[pl.ds(base, CHUNK)], dstw.at[0])
        for j in range(CHUNK // 16):
            sl = pl.ds(j * 16, 16)
            gsrcv[sl] = srcv[sl] * 2 + c
        pltpu.sync_copy(dbh_hbm.at[gsrcv], dbh_b)
        pltpu.sync_copy(eh_hbm.at[dstw.at[0]], ct_b)
        pltpu.sync_copy(ce_hbm.at[pl.ds(base, CHUNK)], ce_b)

        def edge(i, carry2):
            for j in range(HALF // 16):
                sl = pl.ds(j * 16, 16)
                sh = pl.ds(HALF + j * 16, 16)
                dsl = pl.ds(c * HALF + j * 16, 16)
                en = ce_b[i, dsl] + dbh_b[i, sl] + ct_b[i, dsl]
                ce_b[i, dsl] = en
                sg = 1.0 / (1.0 + jnp.exp(-en))
                ct_b[i, sl] = sg * dbh_b[i, sh]
                ct_b[i, sh] = sg
            return carry2

        lax.fori_loop(0, CHUNK, edge, 0)
        pltpu.sync_copy(ce_b, enew_hbm.at[c, pl.ds(base, CHUNK)])
        pltpu.sync_copy(ct_b, acc.at[dstw.at[0]], add=True)
        return carry

    lax.fori_loop(lo, hi, chunk, 0)
    plsc.subcore_barrier()

    def flush_unit(t, carry):
        row = (s + t * NSUB) * ZUNIT
        pltpu.sync_copy(acc.at[pl.ds(row, ZUNIT)], ct_b.at[pl.ds(0, ZUNIT)])
        pltpu.sync_copy(ct_b.at[pl.ds(0, ZUNIT)],
                        nd_hbm.at[c, pl.ds(row, ZUNIT)])
        return carry

    lax.fori_loop(0, nunit, flush_unit, 0)


def _edge_sc(src, dst, dbh_t, eh_t, ce2):
    return pl.kernel(
        _edge_body,
        out_type=[jax.ShapeDtypeStruct((2, N_EDGES, HID), F32),
                  jax.ShapeDtypeStruct((2, N_NODES, HID), F32)],
        mesh=_sc_mesh(),
        scratch_types=[
            pltpu.VMEM((CHUNK,), jnp.int32),
            pltpu.VMEM((CHUNK,), jnp.int32),
            pltpu.VMEM((1, CHUNK), jnp.int32),
            pltpu.VMEM((CHUNK, HID), F32),
            pltpu.VMEM((CHUNK, HID), F32),
            pltpu.VMEM((CHUNK, HID), F32),
            pltpu.VMEM_SHARED((N_NODES, HID), F32),
        ],
    )(src, dst, dbh_t, eh_t, ce2)


def _ro_body(src_hbm, dst_hbm, p_hbm, q_hbm, y1_hbm,
             srcv, dstv, p_b, q_b, y_b):
    c = lax.axis_index("c")
    s = lax.axis_index("s")
    w = c * NSUB + s
    lo = (NCHUNK * w) // (2 * NSUB)
    hi = (NCHUNK * (w + 1)) // (2 * NSUB)

    def chunk(k, carry):
        base = k * CHUNK
        pltpu.sync_copy(src_hbm.at[pl.ds(base, CHUNK)], srcv)
        pltpu.sync_copy(dst_hbm.at[pl.ds(base, CHUNK)], dstv)
        pltpu.sync_copy(p_hbm.at[srcv], p_b)
        pltpu.sync_copy(q_hbm.at[dstv], q_b)

        def edge(i, carry2):
            for j in range(HID // 16):
                sl = pl.ds(j * 16, 16)
                y_b[i, sl] = p_b[i, sl] + q_b[i, sl]
            return carry2

        lax.fori_loop(0, CHUNK, edge, 0)
        pltpu.sync_copy(y_b, y1_hbm.at[pl.ds(base, CHUNK)])
        return carry

    lax.fori_loop(lo, hi, chunk, 0)


def _ro_sc(src, dst, p_t, q_t):
    return pl.kernel(
        _ro_body,
        out_type=jax.ShapeDtypeStruct((N_EDGES, HID), F32),
        mesh=_sc_mesh(),
        scratch_types=[
            pltpu.VMEM((CHUNK,), jnp.int32),
            pltpu.VMEM((CHUNK,), jnp.int32),
            pltpu.VMEM((CHUNK, HID), F32),
            pltpu.VMEM((CHUNK, HID), F32),
            pltpu.VMEM((CHUNK, HID), F32),
        ],
    )(src, dst, p_t, q_t)


def _layer_weights(lp):
    wcat = jnp.concatenate(
        [lp['A_w'], lp['D_w'][:, :HALF], lp['B_w'][:, :HALF],
         lp['D_w'][:, HALF:], lp['B_w'][:, HALF:], lp['E_w']], axis=1)
    bcat = jnp.concatenate(
        [lp['A_b'], lp['D_b'][:HALF], lp['B_b'][:HALF],
         lp['D_b'][HALF:], lp['B_b'][HALF:], lp['E_b']])
    return wcat, bcat


def kernel(h, e, edge_index, params):
    src = edge_index[0]
    dst = edge_index[1]
    p = params

    h = _node_mm(h, p['emb_h_w'], p['emb_h_b'], (HID,))[0]
    e = _edge_mm(e, p['emb_e_w'], p['emb_e_b'])

    s_out = None
    for i, lp in enumerate(p['layers']):
        wcat, bcat = _layer_weights(lp)
        ah, db, e_ = _node_mm(h, wcat, bcat, (HID, 2 * HID, HID))
        dbh_t = db.reshape(2 * N_NODES, HID)
        ce = _edge_mm(e, lp['C_w'], lp['C_b'])
        eh_t = jnp.concatenate([e_, e_], axis=0)
        enew2, nd = _edge_sc(src, dst, dbh_t, eh_t, ce)
        ssum, ssq = _estats(enew2)
        e = _eapply(enew2, e, ssum, ssq, lp['bn_e_g'], lp['bn_e_b'])
        if i == 1:
            h, s_out = _h_fin(ah, nd, h, lp['bn_h_g'], lp['bn_h_b'],
                              lp['S_w'], lp['S_b'])
        else:
            h = _h_fin(ah, nd, h, lp['bn_h_g'], lp['bn_h_b'])

    (w1, b1), (w2, b2), (w3, b3) = p['mlp']
    wr = jnp.concatenate([w1[:HID, :], w1[HID:, :]], axis=1)
    pq = _node_mm(h, wr, jnp.zeros((2 * HID,), F32), (HID, HID))
    y1 = _ro_sc(src, dst, pq[0], pq[1])
    y = _head(y1, b1, w2, b2, w3, b3)
    return (y, s_out)


# speedup vs baseline: 2.9844x; 2.3392x over previous
import functools

import jax
import jax.numpy as jnp
from jax import lax
from jax.experimental import pallas as pl
from jax.experimental.pallas import tpu as pltpu
from jax.experimental.pallas import tpu_sc as plsc

N_NODES = 10000
N_EDGES = 320000
HID = 128
HALF = 64
CHUNK = 128
NCHUNK = N_EDGES // CHUNK
NSUB = 16
ZUNIT = 16
NZUNIT = N_NODES // ZUNIT
EBLK = 4000
NEB = N_EDGES // EBLK
F32 = jnp.float32

@functools.cache
def _sc_mesh():
    return plsc.VectorSubcoreMesh(core_axis_name="c", subcore_axis_name="s",
                                  num_cores=2, num_subcores=NSUB)


def _mm_body(x_ref, w_ref, b_ref, *out_refs, widths):
    y = jnp.dot(x_ref[...], w_ref[...], preferred_element_type=F32) + b_ref[...]
    off = 0
    for r, w in zip(out_refs, widths):
        r[...] = y[:, off:off + w]
        off += w


def _node_mm(x, w, b, widths):
    n = x.shape[0]
    out = pl.pallas_call(
        functools.partial(_mm_body, widths=widths),
        out_shape=[jax.ShapeDtypeStruct((n, wd), F32) for wd in widths],
    )(x, w, b.reshape(1, -1))
    return out


def _emm_body(x_ref, w_ref, b_ref, o_ref):
    o_ref[...] = (jnp.dot(x_ref[...], w_ref[...], preferred_element_type=F32)
                  + b_ref[...])


def _edge_mm(x, w, b):
    din, dout = w.shape
    return pl.pallas_call(
        _emm_body,
        grid=(NEB,),
        in_specs=[pl.BlockSpec((EBLK, din), lambda j: (j, 0)),
                  pl.BlockSpec((din, dout), lambda j: (0, 0)),
                  pl.BlockSpec((1, dout), lambda j: (0, 0))],
        out_specs=pl.BlockSpec((EBLK, dout), lambda j: (j, 0)),
        out_shape=jax.ShapeDtypeStruct((N_EDGES, dout), F32),
    )(x, w, b.reshape(1, -1))


def _estats_body(en_ref, sum_ref, sq_ref):
    j = pl.program_id(0)
    en = jnp.concatenate([en_ref[0, :, :HALF], en_ref[1, :, HALF:]], axis=-1)

    @pl.when(j == 0)
    def _():
        sum_ref[...] = jnp.zeros_like(sum_ref)
        sq_ref[...] = jnp.zeros_like(sq_ref)

    sum_ref[...] += jnp.sum(en, axis=0, keepdims=True)
    sq_ref[...] += jnp.sum(en * en, axis=0, keepdims=True)


def _estats(enew2):
    return pl.pallas_call(
        _estats_body,
        grid=(NEB,),
        in_specs=[pl.BlockSpec((2, EBLK, HID), lambda j: (0, j, 0))],
        out_specs=[pl.BlockSpec((1, HID), lambda j: (0, 0)),
                   pl.BlockSpec((1, HID), lambda j: (0, 0))],
        out_shape=[jax.ShapeDtypeStruct((1, HID), F32),
                   jax.ShapeDtypeStruct((1, HID), F32)],
    )(enew2)


def _eapply_body(en_ref, ein_ref, sum_ref, sq_ref, g_ref, b_ref, o_ref):
    en = jnp.concatenate([en_ref[0, :, :HALF], en_ref[1, :, HALF:]], axis=-1)
    mu = sum_ref[...] / N_EDGES
    var = sq_ref[...] / N_EDGES - mu * mu
    xn = g_ref[...] * (en - mu) / jnp.sqrt(var + 1e-5) + b_ref[...]
    o_ref[...] = ein_ref[...] + jnp.maximum(xn, 0.0)


def _eapply(enew2, e_in, ssum, ssq, g, b):
    return pl.pallas_call(
        _eapply_body,
        grid=(NEB,),
        in_specs=[pl.BlockSpec((2, EBLK, HID), lambda j: (0, j, 0)),
                  pl.BlockSpec((EBLK, HID), lambda j: (j, 0)),
                  pl.BlockSpec((1, HID), lambda j: (0, 0)),
                  pl.BlockSpec((1, HID), lambda j: (0, 0)),
                  pl.BlockSpec((1, HID), lambda j: (0, 0)),
                  pl.BlockSpec((1, HID), lambda j: (0, 0))],
        out_specs=pl.BlockSpec((EBLK, HID), lambda j: (j, 0)),
        out_shape=jax.ShapeDtypeStruct((N_EDGES, HID), F32),
    )(enew2, e_in, ssum, ssq, g.reshape(1, -1), b.reshape(1, -1))


def _hfin_body(ah_ref, nd_ref, hin_ref, g_ref, b_ref, *rest, assign):
    num = jnp.concatenate([nd_ref[0, :, :HALF], nd_ref[1, :, :HALF]], axis=-1)
    den = jnp.concatenate([nd_ref[0, :, HALF:], nd_ref[1, :, HALF:]], axis=-1)
    hn = ah_ref[...] + num / (den + 1e-6)
    mu = jnp.mean(hn, axis=0, keepdims=True)
    var = jnp.mean((hn - mu) ** 2, axis=0, keepdims=True)
    hn = g_ref[...] * (hn - mu) / jnp.sqrt(var + 1e-5) + b_ref[...]
    hn = hin_ref[...] + jnp.maximum(hn, 0.0)
    if assign:
        sw_ref, sb_ref, o_ref, s_ref = rest
        o_ref[...] = hn
        logits = jnp.dot(hn, sw_ref[...], preferred_element_type=F32) + sb_ref[...]
        s_ref[...] = jax.nn.softmax(logits, axis=-1)
    else:
        (o_ref,) = rest
        o_ref[...] = hn


def _h_fin(ah, nd, h_in, g, b, sw=None, sb=None):
    assign = sw is not None
    if assign:
        out_shape = [jax.ShapeDtypeStruct((N_NODES, HID), F32),
                     jax.ShapeDtypeStruct((N_NODES, sw.shape[1]), F32)]
        args = (ah, nd, h_in, g.reshape(1, -1), b.reshape(1, -1),
                sw, sb.reshape(1, -1))
    else:
        out_shape = jax.ShapeDtypeStruct((N_NODES, HID), F32)
        args = (ah, nd, h_in, g.reshape(1, -1), b.reshape(1, -1))
    return pl.pallas_call(
        functools.partial(_hfin_body, assign=assign),
        out_shape=out_shape,
    )(*args)


def _head_body(y1_ref, b1_ref, w2_ref, b2_ref, w3_ref, b3_ref, o_ref):
    y = y1_ref[...] + b1_ref[...]
    y = jnp.maximum(y, 0.0)
    y = jnp.maximum(jnp.dot(y, w2_ref[...], preferred_element_type=F32)
                    + b2_ref[...], 0.0)
    o_ref[...] = jnp.dot(y, w3_ref[...], preferred_element_type=F32) + b3_ref[...]


def _head(y1, b1, w2, b2, w3, b3):
    d2 = w2.shape[1]
    do = w3.shape[1]
    return pl.pallas_call(
        _head_body,
        grid=(NEB,),
        in_specs=[pl.BlockSpec((EBLK, HID), lambda j: (j, 0)),
                  pl.BlockSpec((1, HID), lambda j: (0, 0)),
                  pl.BlockSpec((HID, d2), lambda j: (0, 0)),
                  pl.BlockSpec((1, d2), lambda j: (0, 0)),
                  pl.BlockSpec((d2, do), lambda j: (0, 0)),
                  pl.BlockSpec((1, do), lambda j: (0, 0))],
        out_specs=pl.BlockSpec((EBLK, do), lambda j: (j, 0)),
        out_shape=jax.ShapeDtypeStruct((N_EDGES, do), F32),
    )(y1, b1.reshape(1, -1), w2, b2.reshape(1, -1), w3, b3.reshape(1, -1))


def _edge_body(src_hbm, dst_hbm, dbh_hbm, eh_hbm, ce_hbm,
               enew_hbm, nd_hbm,
               srcv, gsrcv, dstw, ce_b, dbh_b, ct_b, acc, sem_i, sem_b):
    c = lax.axis_index("c")
    s = lax.axis_index("s")
    nunit = (NZUNIT - s + NSUB - 1) // NSUB

    def zrow(r, carry):
        for j in range(HID // 16):
            ct_b[r, pl.ds(j * 16, 16)] = jnp.zeros((16,), F32)
        return carry

    lax.fori_loop(0, ZUNIT, zrow, 0)

    def zero_unit(t, carry):
        row = (s + t * NSUB) * ZUNIT
        pltpu.sync_copy(ct_b.at[pl.ds(0, ZUNIT)], acc.at[pl.ds(row, ZUNIT)])
        return carry

    lax.fori_loop(0, nunit, zero_unit, 0)
    plsc.subcore_barrier()
    lo = (NCHUNK * s) // NSUB
    hi = (NCHUNK * (s + 1)) // NSUB

    def chunk(k, carry):
        base = k * CHUNK
        cp_s = pltpu.async_copy(src_hbm.at[pl.ds(base, CHUNK)], srcv, sem_i)
        cp_d = pltpu.async_copy(dst_hbm.at[pl.ds(base, CHUNK)], dstw.at[0],
                                sem_i)
        cp_c = pltpu.async_copy(ce_hbm.at[pl.ds(base, CHUNK)], ce_b, sem_b)
        cp_s.wait()
        cp_d.wait()
        for j in range(CHUNK // 16):
            sl = pl.ds(j * 16, 16)
            gsrcv[sl] = srcv[sl] * 2 + c
        cp_g = pltpu.async_copy(dbh_hbm.at[gsrcv], dbh_b, sem_b)
        cp_e = pltpu.async_copy(eh_hbm.at[dstw.at[0]], ct_b, sem_b)
        cp_c.wait()
        cp_g.wait()
        cp_e.wait()

        @plsc.parallel_loop(0, CHUNK, 1, unroll=4)
        def edge(i):
            ehs = [ct_b[i, pl.ds(c * HALF + j * 16, 16)]
                   for j in range(HALF // 16)]
            for j in range(HALF // 16):
                sl = pl.ds(j * 16, 16)
                sh = pl.ds(HALF + j * 16, 16)
                dsl = pl.ds(c * HALF + j * 16, 16)
                en = ce_b[i, dsl] + dbh_b[i, sl] + ehs[j]
                ce_b[i, dsl] = en
                sg = 1.0 / (1.0 + jnp.exp(-en))
                ct_b[i, sl] = sg * dbh_b[i, sh]
                ct_b[i, sh] = sg

        pltpu.sync_copy(ce_b, enew_hbm.at[c, pl.ds(base, CHUNK)])
        pltpu.sync_copy(ct_b, acc.at[dstw.at[0]], add=True)
        return carry

    lax.fori_loop(lo, hi, chunk, 0)
    plsc.subcore_barrier()

    def flush_unit(t, carry):
        row = (s + t * NSUB) * ZUNIT
        pltpu.sync_copy(acc.at[pl.ds(row, ZUNIT)], ct_b.at[pl.ds(0, ZUNIT)])
        pltpu.sync_copy(ct_b.at[pl.ds(0, ZUNIT)],
                        nd_hbm.at[c, pl.ds(row, ZUNIT)])
        return carry

    lax.fori_loop(0, nunit, flush_unit, 0)


def _edge_sc(src, dst, dbh_t, eh_t, ce2):
    return pl.kernel(
        _edge_body,
        out_type=[jax.ShapeDtypeStruct((2, N_EDGES, HID), F32),
                  jax.ShapeDtypeStruct((2, N_NODES, HID), F32)],
        mesh=_sc_mesh(),
        scratch_types=[
            pltpu.VMEM((CHUNK,), jnp.int32),
            pltpu.VMEM((CHUNK,), jnp.int32),
            pltpu.VMEM((1, CHUNK), jnp.int32),
            pltpu.VMEM((CHUNK, HID), F32),
            pltpu.VMEM((CHUNK, HID), F32),
            pltpu.VMEM((CHUNK, HID), F32),
            pltpu.VMEM_SHARED((N_NODES, HID), F32),
            pltpu.SemaphoreType.DMA,
            pltpu.SemaphoreType.DMA,
        ],
    )(src, dst, dbh_t, eh_t, ce2)


def _ro_body(src_hbm, dst_hbm, p_hbm, q_hbm, y1_hbm,
             srcv, dstv, p_b, q_b, y_b, sem_i, sem_b):
    c = lax.axis_index("c")
    s = lax.axis_index("s")
    w = c * NSUB + s
    lo = (NCHUNK * w) // (2 * NSUB)
    hi = (NCHUNK * (w + 1)) // (2 * NSUB)

    def chunk(k, carry):
        base = k * CHUNK
        cp_s = pltpu.async_copy(src_hbm.at[pl.ds(base, CHUNK)], srcv, sem_i)
        cp_d = pltpu.async_copy(dst_hbm.at[pl.ds(base, CHUNK)], dstv, sem_i)
        cp_s.wait()
        cp_d.wait()
        cp_p = pltpu.async_copy(p_hbm.at[srcv], p_b, sem_b)
        cp_q = pltpu.async_copy(q_hbm.at[dstv], q_b, sem_b)
        cp_p.wait()
        cp_q.wait()

        @plsc.parallel_loop(0, CHUNK, 1, unroll=4)
        def edge(i):
            for j in range(HID // 16):
                sl = pl.ds(j * 16, 16)
                y_b[i, sl] = p_b[i, sl] + q_b[i, sl]

        pltpu.sync_copy(y_b, y1_hbm.at[pl.ds(base, CHUNK)])
        return carry

    lax.fori_loop(lo, hi, chunk, 0)


def _ro_sc(src, dst, p_t, q_t):
    return pl.kernel(
        _ro_body,
        out_type=jax.ShapeDtypeStruct((N_EDGES, HID), F32),
        mesh=_sc_mesh(),
        scratch_types=[
            pltpu.VMEM((CHUNK,), jnp.int32),
            pltpu.VMEM((CHUNK,), jnp.int32),
            pltpu.VMEM((CHUNK, HID), F32),
            pltpu.VMEM((CHUNK, HID), F32),
            pltpu.VMEM((CHUNK, HID), F32),
            pltpu.SemaphoreType.DMA,
            pltpu.SemaphoreType.DMA,
        ],
    )(src, dst, p_t, q_t)


def _layer_weights(lp):
    wcat = jnp.concatenate(
        [lp['A_w'], lp['D_w'][:, :HALF], lp['B_w'][:, :HALF],
         lp['D_w'][:, HALF:], lp['B_w'][:, HALF:], lp['E_w']], axis=1)
    bcat = jnp.concatenate(
        [lp['A_b'], lp['D_b'][:HALF], lp['B_b'][:HALF],
         lp['D_b'][HALF:], lp['B_b'][HALF:], lp['E_b']])
    return wcat, bcat


def kernel(h, e, edge_index, params):
    src = edge_index[0]
    dst = edge_index[1]
    p = params

    h = _node_mm(h, p['emb_h_w'], p['emb_h_b'], (HID,))[0]
    e = _edge_mm(e, p['emb_e_w'], p['emb_e_b'])

    s_out = None
    for i, lp in enumerate(p['layers']):
        wcat, bcat = _layer_weights(lp)
        ah, db, e_ = _node_mm(h, wcat, bcat, (HID, 2 * HID, HID))
        dbh_t = db.reshape(2 * N_NODES, HID)
        ce = _edge_mm(e, lp['C_w'], lp['C_b'])
        eh_t = jnp.concatenate([e_, e_], axis=0)
        enew2, nd = _edge_sc(src, dst, dbh_t, eh_t, ce)
        ssum, ssq = _estats(enew2)
        e = _eapply(enew2, e, ssum, ssq, lp['bn_e_g'], lp['bn_e_b'])
        if i == 1:
            h, s_out = _h_fin(ah, nd, h, lp['bn_h_g'], lp['bn_h_b'],
                              lp['S_w'], lp['S_b'])
        else:
            h = _h_fin(ah, nd, h, lp['bn_h_g'], lp['bn_h_b'])

    (w1, b1), (w2, b2), (w3, b3) = p['mlp']
    wr = jnp.concatenate([w1[:HID, :], w1[HID:, :]], axis=1)
    pq = _node_mm(h, wr, jnp.zeros((2 * HID,), F32), (HID, HID))
    y1 = _ro_sc(src, dst, pq[0], pq[1])
    y = _head(y1, b1, w2, b2, w3, b3)
    return (y, s_out)
